# baseline (speedup 1.0000x reference)
import functools

import jax
import jax.numpy as jnp
from jax import lax
from jax.experimental import pallas as pl
from jax.experimental.pallas import tpu as pltpu
from jax.experimental.pallas import tpu_sc as plsc

NC = 2
NS = 16
NW = NC * NS
NBUF = 4


@functools.partial(jax.jit, static_argnums=(2,))
def _sc_gather(ids, table, shapes):
    b, h, v, d = shapes
    bw = b // NW
    mesh = plsc.VectorSubcoreMesh(core_axis_name="c", subcore_axis_name="s")
    ngroup = bw // NBUF
    assert ngroup * NBUF == bw

    @functools.partial(
        pl.kernel,
        out_type=jax.ShapeDtypeStruct((b, h, d), jnp.float32),
        mesh=mesh,
        scratch_types=[
            pltpu.VMEM((bw, h), jnp.int32),
            pltpu.VMEM((NBUF, h, d), jnp.float32),
            pltpu.SemaphoreType.DMA((NBUF,)),
            pltpu.SemaphoreType.DMA((NBUF,)),
        ],
        compiler_params=pltpu.CompilerParams(use_tc_tiling_on_sc=False),
    )
    def k(ids_hbm, table_hbm, out_hbm, idx_v, rows_v, gsem, wsem):
        wid = lax.axis_index("s") * NC + lax.axis_index("c")
        base = wid * bw
        pltpu.sync_copy(ids_hbm.at[pl.ds(base, bw)], idx_v)

        for bb in range(NBUF):
            pltpu.async_copy(table_hbm.at[idx_v.at[bb]], rows_v.at[bb], gsem.at[bb])

        def group(g, carry):
            for bb in range(NBUF):
                i = g * NBUF + bb
                pltpu.make_async_copy(
                    table_hbm.at[idx_v.at[i]], rows_v.at[bb], gsem.at[bb]
                ).wait()
                pltpu.async_copy(rows_v.at[bb], out_hbm.at[base + i], wsem.at[bb])

                @pl.when(g < ngroup - 1)
                def _():
                    pltpu.make_async_copy(
                        rows_v.at[bb], out_hbm.at[base + i], wsem.at[bb]
                    ).wait()
                    pltpu.async_copy(
                        table_hbm.at[idx_v.at[i + NBUF]], rows_v.at[bb], gsem.at[bb]
                    )

            return carry

        lax.fori_loop(0, ngroup, group, 0)

        for bb in range(NBUF):
            i = (ngroup - 1) * NBUF + bb
            pltpu.make_async_copy(
                rows_v.at[bb], out_hbm.at[base + i], wsem.at[bb]
            ).wait()

    return k(ids, table)


def kernel(input_ids, table):
    b, h = input_ids.shape
    v, d = table.shape
    assert b % NW == 0
    out = _sc_gather(input_ids.astype(jnp.int32), table, (b, h, v, d))
    return out[..., None, None]


# speedup vs baseline: 1.0027x; 1.0027x over previous
import functools

import jax
import jax.numpy as jnp
from jax import lax
from jax.experimental import pallas as pl
from jax.experimental.pallas import tpu as pltpu
from jax.experimental.pallas import tpu_sc as plsc

NC = 2
NS = 16
NW = NC * NS
NBUF = 8


@functools.partial(jax.jit, static_argnums=(2,))
def _sc_gather(ids, table, shapes):
    b, h, v, d = shapes
    bw = b // NW
    mesh = plsc.VectorSubcoreMesh(core_axis_name="c", subcore_axis_name="s")
    ngroup = bw // NBUF
    assert ngroup * NBUF == bw

    @functools.partial(
        pl.kernel,
        out_type=jax.ShapeDtypeStruct((b, h, d), jnp.float32),
        mesh=mesh,
        scratch_types=[
            pltpu.VMEM((bw, h), jnp.int32),
            pltpu.VMEM((NBUF, h, d), jnp.float32),
            pltpu.SemaphoreType.DMA((NBUF,)),
            pltpu.SemaphoreType.DMA((NBUF,)),
        ],
        compiler_params=pltpu.CompilerParams(use_tc_tiling_on_sc=False),
    )
    def k(ids_hbm, table_hbm, out_hbm, idx_v, rows_v, gsem, wsem):
        wid = lax.axis_index("s") * NC + lax.axis_index("c")
        base = wid * bw
        pltpu.sync_copy(ids_hbm.at[pl.ds(base, bw)], idx_v)

        for bb in range(NBUF):
            pltpu.async_copy(table_hbm.at[idx_v.at[bb]], rows_v.at[bb], gsem.at[bb])

        def group(g, carry):
            for bb in range(NBUF):
                i = g * NBUF + bb
                pltpu.make_async_copy(
                    table_hbm.at[idx_v.at[i]], rows_v.at[bb], gsem.at[bb]
                ).wait()
                pltpu.async_copy(rows_v.at[bb], out_hbm.at[base + i], wsem.at[bb])

                @pl.when(g < ngroup - 1)
                def _():
                    pltpu.make_async_copy(
                        rows_v.at[bb], out_hbm.at[base + i], wsem.at[bb]
                    ).wait()
                    pltpu.async_copy(
                        table_hbm.at[idx_v.at[i + NBUF]], rows_v.at[bb], gsem.at[bb]
                    )

            return carry

        lax.fori_loop(0, ngroup, group, 0)

        for bb in range(NBUF):
            i = (ngroup - 1) * NBUF + bb
            pltpu.make_async_copy(
                rows_v.at[bb], out_hbm.at[base + i], wsem.at[bb]
            ).wait()

    return k(ids, table)


def kernel(input_ids, table):
    b, h = input_ids.shape
    v, d = table.shape
    assert b % NW == 0
    out = _sc_gather(input_ids.astype(jnp.int32), table, (b, h, v, d))
    return out[..., None, None]
